# baseline (speedup 1.0000x reference)
import functools

import jax
import jax.numpy as jnp
from jax import lax
from jax.experimental import pallas as pl
from jax.experimental.pallas import tpu as pltpu
from jax.experimental.pallas import tpu_sc as plsc

D = 128
CHUNK = 128
NC = 2
NS = 16
NW = NC * NS
IDX_STAGE = 40
NBUF = 4


@functools.lru_cache(maxsize=None)
def _make_sc_embed(n_rows: int):
    rows_per_w = n_rows // NW
    n_stages = rows_per_w // IDX_STAGE
    groups_per_stage = IDX_STAGE // NBUF

    mesh = plsc.VectorSubcoreMesh(core_axis_name="c", subcore_axis_name="s")

    @functools.partial(
        pl.kernel,
        mesh=mesh,
        out_type=jax.ShapeDtypeStruct((n_rows * CHUNK, D), jnp.float32),
        scratch_types=(
            [pltpu.VMEM((IDX_STAGE, CHUNK), jnp.int32)]
            + [pltpu.VMEM((CHUNK, D), jnp.float32) for _ in range(NBUF)]
            + [pltpu.SemaphoreType.DMA for _ in range(2 * NBUF)]
        ),
    )
    def sc_embed(x_hbm, w_hbm, out_hbm, idx_v, *bufs_and_sems):
        rows = bufs_and_sems[:NBUF]
        gsems = bufs_and_sems[NBUF : 2 * NBUF]
        wsems = bufs_and_sems[2 * NBUF : 3 * NBUF]
        cid = lax.axis_index("c")
        sid = lax.axis_index("s")
        wid = sid * NC + cid
        row0 = wid * rows_per_w

        def stage_body(st, carry):
            srow = row0 + st * IDX_STAGE
            pltpu.sync_copy(x_hbm.at[pl.ds(srow, IDX_STAGE)], idx_v)

            def group_body(g, carry):
                grow = srow + g * NBUF
                chunk0 = st * IDX_STAGE + g * NBUF
                gathers = []
                for b in range(NBUF):
                    @pl.when(chunk0 + b >= NBUF)
                    def _wait_prev_write():
                        pltpu.make_async_copy(
                            rows[b], out_hbm.at[pl.ds(0, CHUNK)], wsems[b]
                        ).wait()

                    cp = pltpu.make_async_copy(
                        w_hbm.at[idx_v.at[g * NBUF + b]], rows[b], gsems[b]
                    )
                    cp.start()
                    gathers.append(cp)
                for b in range(NBUF):
                    gathers[b].wait()
                    pltpu.make_async_copy(
                        rows[b],
                        out_hbm.at[pl.ds((grow + b) * CHUNK, CHUNK)],
                        wsems[b],
                    ).start()
                return carry

            return lax.fori_loop(0, groups_per_stage, group_body, carry)

        lax.fori_loop(0, n_stages, stage_body, 0)
        for b in range(NBUF):
            pltpu.make_async_copy(
                rows[b], out_hbm.at[pl.ds(0, CHUNK)], wsems[b]
            ).wait()

    return sc_embed


def kernel(x, weight):
    n, m = x.shape
    total = n * m
    x2 = x.astype(jnp.int32).reshape(total // CHUNK, CHUNK)
    out = _make_sc_embed(total // CHUNK)(x2, weight)
    return out.reshape(n, m, D)


# speedup vs baseline: 35.1330x; 35.1330x over previous
import functools

import jax
import jax.numpy as jnp
from jax import lax
from jax.experimental import pallas as pl
from jax.experimental.pallas import tpu as pltpu
from jax.experimental.pallas import tpu_sc as plsc

D = 128
CHUNK = 128
NC = 2
NS = 16
NW = NC * NS
IDX_STAGE = 40
NBUF = 4


@functools.lru_cache(maxsize=None)
def _make_sc_embed(n_rows: int):
    rows_per_w = n_rows // NW
    n_stages = rows_per_w // IDX_STAGE
    groups_per_stage = IDX_STAGE // NBUF

    mesh = plsc.VectorSubcoreMesh(core_axis_name="c", subcore_axis_name="s")

    @functools.partial(
        pl.kernel,
        mesh=mesh,
        out_type=jax.ShapeDtypeStruct((n_rows * CHUNK, D), jnp.float32),
        scratch_types=(
            [pltpu.VMEM((IDX_STAGE, CHUNK), jnp.int32)]
            + [pltpu.VMEM_SHARED((5, D), jnp.float32)]
            + [pltpu.VMEM((CHUNK, D), jnp.float32) for _ in range(NBUF)]
            + [pltpu.SemaphoreType.DMA for _ in range(2 * NBUF)]
        ),
    )
    def sc_embed(x_hbm, w_hbm, out_hbm, idx_v, w_v, *bufs_and_sems):
        rows = bufs_and_sems[:NBUF]
        gsems = bufs_and_sems[NBUF : 2 * NBUF]
        wsems = bufs_and_sems[2 * NBUF : 3 * NBUF]
        cid = lax.axis_index("c")
        sid = lax.axis_index("s")
        wid = sid * NC + cid
        row0 = wid * rows_per_w
        @pl.when(sid == 0)
        def _stage_table():
            pltpu.sync_copy(w_hbm, w_v)

        plsc.subcore_barrier()

        def stage_body(st, carry):
            srow = row0 + st * IDX_STAGE
            pltpu.sync_copy(x_hbm.at[pl.ds(srow, IDX_STAGE)], idx_v)

            def group_body(g, carry):
                grow = srow + g * NBUF
                chunk0 = st * IDX_STAGE + g * NBUF
                gathers = []
                for b in range(NBUF):
                    @pl.when(chunk0 + b >= NBUF)
                    def _wait_prev_write():
                        pltpu.make_async_copy(
                            rows[b], out_hbm.at[pl.ds(0, CHUNK)], wsems[b]
                        ).wait()

                    cp = pltpu.make_async_copy(
                        w_v.at[idx_v.at[g * NBUF + b]], rows[b], gsems[b]
                    )
                    cp.start()
                    gathers.append(cp)
                for b in range(NBUF):
                    gathers[b].wait()
                    pltpu.make_async_copy(
                        rows[b],
                        out_hbm.at[pl.ds((grow + b) * CHUNK, CHUNK)],
                        wsems[b],
                    ).start()
                return carry

            return lax.fori_loop(0, groups_per_stage, group_body, carry)

        lax.fori_loop(0, n_stages, stage_body, 0)
        for b in range(NBUF):
            pltpu.make_async_copy(
                rows[b], out_hbm.at[pl.ds(0, CHUNK)], wsems[b]
            ).wait()

    return sc_embed


def kernel(x, weight):
    n, m = x.shape
    total = n * m
    x2 = x.astype(jnp.int32).reshape(total // CHUNK, CHUNK)
    out = _make_sc_embed(total // CHUNK)(x2, weight)
    return out.reshape(n, m, D)
